# baseline (speedup 1.0000x reference)
import functools

import jax
import jax.numpy as jnp
from jax import lax
from jax.experimental import pallas as pl
from jax.experimental.pallas import tpu as pltpu
from jax.experimental.pallas import tpu_sc as plsc

B = 4096
VOCAB = 1000000
HIST = 50
HP = 52
D = 32
NC = 2
NS = 16
NW = NC * NS
BPW = B // NW
CB = 2
NCHUNK = BPW // CB
CIDX = CB * HP
NBUF = 4


def _splat(vreg, lane):
    idx = jnp.full((16, 1), lane, jnp.int32)
    dnums = lax.GatherDimensionNumbers(
        offset_dims=(), collapsed_slice_dims=(0,), start_index_map=(0,))
    return lax.gather(vreg, idx, dnums, slice_sizes=(1,),
                      mode=lax.GatherScatterMode.PROMISE_IN_BOUNDS)


def _body(ids_hbm, vals_hbm, table_hbm, out_hbm,
          ids_v, vals_v, rows, out_v, sems):
    wid = lax.axis_index("s") * NC + lax.axis_index("c")
    pltpu.sync_copy(ids_hbm.at[pl.ds(wid * (BPW * HP), BPW * HP)], ids_v)
    pltpu.sync_copy(vals_hbm.at[pl.ds(wid * BPW, BPW)], vals_v)

    def start(c, p):
        pltpu.async_copy(
            table_hbm.at[ids_v.at[pl.ds(c * CIDX, CIDX)]], rows[p], sems[p])

    def wait(p):
        pltpu.make_async_copy(
            table_hbm.at[ids_v.at[pl.ds(0, CIDX)]], rows[p], sems[p]).wait()

    for p in range(NBUF):
        start(p, p)

    @pl.loop(0, NCHUNK, step=NBUF)
    def _(i):
        for p in range(NBUF):
            c = i + p
            wait(p)
            for r in range(CB):
                b = c * CB + r
                v0 = vals_v[b, pl.ds(0, 16)]
                v1 = vals_v[b, pl.ds(16, 16)]
                v2 = vals_v[b, pl.ds(32, 16)]
                v3 = vals_v[b, pl.ds(36, 16)]
                acc0 = jnp.zeros((16,), jnp.float32)
                acc1 = jnp.zeros((16,), jnp.float32)
                for l in range(HP):
                    if l < 48:
                        w = _splat((v0, v1, v2)[l // 16], l % 16)
                    else:
                        w = _splat(v3, l - 36)
                    e0 = rows[p][r * HP + l, pl.ds(0, 16)]
                    e1 = rows[p][r * HP + l, pl.ds(16, 16)]
                    acc0 = acc0 + w * e0
                    acc1 = acc1 + w * e1
                out_v[b, pl.ds(0, 16)] = acc0
                out_v[b, pl.ds(16, 16)] = acc1

            @pl.when(c + NBUF < NCHUNK)
            def _():
                start(c + NBUF, p)

    pltpu.sync_copy(out_v, out_hbm.at[pl.ds(wid * BPW, BPW)])


def _entry(ids_flat, vals_pad, table, out,
           ids_v, vals_v, r0, r1, r2, r3, out_v, s0, s1, s2, s3):
    _body(ids_flat, vals_pad, table, out,
          ids_v, vals_v, (r0, r1, r2, r3), out_v, (s0, s1, s2, s3))


_sc_call = pl.kernel(
    _entry,
    out_type=jax.ShapeDtypeStruct((B, D), jnp.float32),
    mesh=plsc.VectorSubcoreMesh(core_axis_name="c", subcore_axis_name="s"),
    scratch_types=[
        pltpu.VMEM((BPW * HP,), jnp.int32),
        pltpu.VMEM((BPW, HP), jnp.float32),
        *[pltpu.VMEM((CIDX, D), jnp.float32) for _ in range(NBUF)],
        pltpu.VMEM((BPW, D), jnp.float32),
        *[pltpu.SemaphoreType.DMA for _ in range(NBUF)],
    ],
    compiler_params=pltpu.CompilerParams(use_tc_tiling_on_sc=False),
)


GCOL = 512
NFULL = 999936 // GCOL
VTAIL = VOCAB - 999936
IPW = 62


def _transpose_block(src, dst_flat):
    iota16 = lax.iota(jnp.int32, 16)

    @pl.loop(0, GCOL, step=8)
    def _(c0):
        for u in range(8):
            c = c0 + u
            cvec = jnp.full((16,), 0, jnp.int32) + c
            x0 = plsc.load_gather(src, [iota16, cvec])
            x1 = plsc.load_gather(src, [iota16 + 16, cvec])
            dst_flat[pl.ds(c * D, 16)] = x0
            dst_flat[pl.ds(c * D + 16, 16)] = x1


def _relayout_entry(pt_hbm, tail_hbm, tbl_hbm, ibuf0, ibuf1, obuf0, obuf1,
                    si0, si1, so0, so1):
    wid = lax.axis_index("s") * NC + lax.axis_index("c")
    ibuf = (ibuf0, ibuf1)
    obuf = (obuf0, obuf1)
    sin = (si0, si1)
    sout = (so0, so1)

    def start(g, p):
        for t in range(4):
            pltpu.async_copy(
                pt_hbm.at[pl.ds(8 * t, 8), pl.ds(g * GCOL, GCOL)],
                ibuf[p].at[pl.ds(8 * t, 8), pl.ds(0, GCOL)], sin[p])

    def wait(p):
        for t in range(4):
            pltpu.make_async_copy(
                pt_hbm.at[pl.ds(0, 8), pl.ds(0, GCOL)],
                ibuf[p].at[pl.ds(8 * t, 8), pl.ds(0, GCOL)], sin[p]).wait()

    def wait_out(p):
        pltpu.make_async_copy(
            obuf[p], tbl_hbm.at[pl.ds(0, GCOL * D)], sout[p]).wait()

    start(wid, 0)

    @pl.when(NW + wid < NFULL)
    def _():
        start(NW + wid, 1)

    @pl.loop(0, IPW, step=2)
    def _(i):
        for p in range(2):
            g = (i + p) * NW + wid

            @pl.when(g < NFULL)
            def _():
                wait(p)

                @pl.when(g >= 2 * NW)
                def _():
                    wait_out(p)
                _transpose_block(ibuf[p], obuf[p])
                pltpu.async_copy(
                    obuf[p], tbl_hbm.at[pl.ds(g * (GCOL * D), GCOL * D)],
                    sout[p])
                nxt = g + 2 * NW

                @pl.when(nxt < NFULL)
                def _():
                    start(nxt, p)

    wait_out(0)
    wait_out(1)

    @pl.when(wid == 0)
    def _():
        pltpu.sync_copy(tail_hbm, obuf0.at[pl.ds(0, VTAIL * D)])
        pltpu.sync_copy(obuf0.at[pl.ds(0, VTAIL * D)],
                        tbl_hbm.at[pl.ds(NFULL * (GCOL * D), VTAIL * D)])


_relayout = pl.kernel(
    _relayout_entry,
    out_type=jax.ShapeDtypeStruct((VOCAB * D,), jnp.float32),
    mesh=plsc.VectorSubcoreMesh(core_axis_name="c", subcore_axis_name="s"),
    scratch_types=[
        pltpu.VMEM((D, GCOL), jnp.float32),
        pltpu.VMEM((D, GCOL), jnp.float32),
        pltpu.VMEM((GCOL * D,), jnp.float32),
        pltpu.VMEM((GCOL * D,), jnp.float32),
        pltpu.SemaphoreType.DMA,
        pltpu.SemaphoreType.DMA,
        pltpu.SemaphoreType.DMA,
        pltpu.SemaphoreType.DMA,
    ],
    compiler_params=pltpu.CompilerParams(use_tc_tiling_on_sc=True,
                                         needs_layout_passes=False),
)


@jax.jit
def _run(ids, values, params):
    ids_p = jnp.zeros((B, HP), jnp.int32).at[:, :HIST].set(
        ids.astype(jnp.int32)).reshape(-1)
    vals_p = jnp.zeros((B, HP), jnp.float32).at[:, :HIST].set(values)
    tail = params[NFULL * GCOL:, :].reshape(-1)
    table = _relayout(params.T, tail).reshape(VOCAB, D)
    return _sc_call(ids_p, vals_p, table)


def kernel(ids, values, params):
    return _run(ids, values, params)


# speedup vs baseline: 1.3105x; 1.3105x over previous
import functools

import jax
import jax.numpy as jnp
from jax import lax
from jax.experimental import pallas as pl
from jax.experimental.pallas import tpu as pltpu
from jax.experimental.pallas import tpu_sc as plsc

B = 4096
VOCAB = 1000000
HIST = 50
HP = 52
D = 32
NC = 2
NS = 16
NW = NC * NS
BPW = B // NW
CB = 2
NCHUNK = BPW // CB
CIDX = CB * HP
NBUF = 8


def _splat(vreg, lane):
    idx = jnp.full((16, 1), lane, jnp.int32)
    dnums = lax.GatherDimensionNumbers(
        offset_dims=(), collapsed_slice_dims=(0,), start_index_map=(0,))
    return lax.gather(vreg, idx, dnums, slice_sizes=(1,),
                      mode=lax.GatherScatterMode.PROMISE_IN_BOUNDS)


def _body(ids_hbm, vals_hbm, table_hbm, out_hbm,
          ids_v, vals_v, rows, out_v, sems):
    wid = lax.axis_index("s") * NC + lax.axis_index("c")
    pltpu.sync_copy(ids_hbm.at[pl.ds(wid * (BPW * HP), BPW * HP)], ids_v)
    pltpu.sync_copy(vals_hbm.at[pl.ds(wid * BPW, BPW)], vals_v)

    def start(c, p):
        pltpu.async_copy(
            table_hbm.at[ids_v.at[pl.ds(c * CIDX, CIDX)]], rows[p], sems[p])

    def wait(p):
        pltpu.make_async_copy(
            table_hbm.at[ids_v.at[pl.ds(0, CIDX)]], rows[p], sems[p]).wait()

    for p in range(NBUF):
        start(p, p)

    @pl.loop(0, NCHUNK, step=NBUF)
    def _(i):
        for p in range(NBUF):
            c = i + p
            wait(p)
            for r in range(CB):
                b = c * CB + r
                v0 = vals_v[b, pl.ds(0, 16)]
                v1 = vals_v[b, pl.ds(16, 16)]
                v2 = vals_v[b, pl.ds(32, 16)]
                v3 = vals_v[b, pl.ds(36, 16)]
                acc0 = jnp.zeros((16,), jnp.float32)
                acc1 = jnp.zeros((16,), jnp.float32)
                for l in range(HP):
                    if l < 48:
                        w = _splat((v0, v1, v2)[l // 16], l % 16)
                    else:
                        w = _splat(v3, l - 36)
                    e0 = rows[p][r * HP + l, pl.ds(0, 16)]
                    e1 = rows[p][r * HP + l, pl.ds(16, 16)]
                    acc0 = acc0 + w * e0
                    acc1 = acc1 + w * e1
                out_v[b, pl.ds(0, 16)] = acc0
                out_v[b, pl.ds(16, 16)] = acc1

            @pl.when(c + NBUF < NCHUNK)
            def _():
                start(c + NBUF, p)

    pltpu.sync_copy(out_v, out_hbm.at[pl.ds(wid * BPW, BPW)])


def _entry(ids_flat, vals_pad, table, out, ids_v, vals_v, *rest):
    rows = rest[:NBUF]
    out_v = rest[NBUF]
    sems = rest[NBUF + 1:]
    _body(ids_flat, vals_pad, table, out, ids_v, vals_v, rows, out_v, sems)


_sc_call = pl.kernel(
    _entry,
    out_type=jax.ShapeDtypeStruct((B, D), jnp.float32),
    mesh=plsc.VectorSubcoreMesh(core_axis_name="c", subcore_axis_name="s"),
    scratch_types=[
        pltpu.VMEM((BPW * HP,), jnp.int32),
        pltpu.VMEM((BPW, HP), jnp.float32),
        *[pltpu.VMEM((CIDX, D), jnp.float32) for _ in range(NBUF)],
        pltpu.VMEM((BPW, D), jnp.float32),
        *[pltpu.SemaphoreType.DMA for _ in range(NBUF)],
    ],
    compiler_params=pltpu.CompilerParams(use_tc_tiling_on_sc=False),
)


@jax.jit
def _run(ids, values, params):
    ids_p = jnp.zeros((B, HP), jnp.int32).at[:, :HIST].set(
        ids.astype(jnp.int32)).reshape(-1)
    vals_p = jnp.zeros((B, HP), jnp.float32).at[:, :HIST].set(values)
    return _sc_call(ids_p, vals_p, params)


def kernel(ids, values, params):
    return _run(ids, values, params)


# speedup vs baseline: 1.3196x; 1.0069x over previous
import functools

import jax
import jax.numpy as jnp
from jax import lax
from jax.experimental import pallas as pl
from jax.experimental.pallas import tpu as pltpu
from jax.experimental.pallas import tpu_sc as plsc

B = 4096
VOCAB = 1000000
HIST = 50
HP = 52
D = 32
NC = 2
NS = 16
NW = NC * NS
BPW = B // NW
CB = 2
NCHUNK = BPW // CB
CIDX = CB * HP
NBUF = 4


def _splat(vreg, lane):
    idx = jnp.full((16, 1), lane, jnp.int32)
    dnums = lax.GatherDimensionNumbers(
        offset_dims=(), collapsed_slice_dims=(0,), start_index_map=(0,))
    return lax.gather(vreg, idx, dnums, slice_sizes=(1,),
                      mode=lax.GatherScatterMode.PROMISE_IN_BOUNDS)


def _body(ids_hbm, vals_hbm, table_hbm, out_hbm,
          ids_v, vals_v, rows, out_v, sems):
    wid = lax.axis_index("s") * NC + lax.axis_index("c")
    pltpu.sync_copy(ids_hbm.at[pl.ds(wid * (BPW * HP), BPW * HP)], ids_v)
    pltpu.sync_copy(vals_hbm.at[pl.ds(wid * BPW, BPW)], vals_v)

    def start(c, p):
        pltpu.async_copy(
            table_hbm.at[ids_v.at[pl.ds(c * CIDX, CIDX)]], rows[p], sems[p])

    def wait(p):
        pltpu.make_async_copy(
            table_hbm.at[ids_v.at[pl.ds(0, CIDX)]], rows[p], sems[p]).wait()

    for p in range(NBUF):
        start(p, p)

    @pl.loop(0, NCHUNK, step=NBUF)
    def _(i):
        for p in range(NBUF):
            c = i + p
            wait(p)
            for r in range(CB):
                b = c * CB + r
                v0 = vals_v[b, pl.ds(0, 16)]
                v1 = vals_v[b, pl.ds(16, 16)]
                v2 = vals_v[b, pl.ds(32, 16)]
                v3 = vals_v[b, pl.ds(36, 16)]
                acc0 = jnp.zeros((16,), jnp.float32)
                acc1 = jnp.zeros((16,), jnp.float32)
                for l in range(HP):
                    if l < 48:
                        w = _splat((v0, v1, v2)[l // 16], l % 16)
                    else:
                        w = _splat(v3, l - 36)
                    e0 = rows[p][r * HP + l, pl.ds(0, 16)]
                    e1 = rows[p][r * HP + l, pl.ds(16, 16)]
                    acc0 = acc0 + w * e0
                    acc1 = acc1 + w * e1
                out_v[b, pl.ds(0, 16)] = acc0
                out_v[b, pl.ds(16, 16)] = acc1

            @pl.when(c + NBUF < NCHUNK)
            def _():
                start(c + NBUF, p)

    pltpu.sync_copy(out_v, out_hbm.at[pl.ds(wid * BPW, BPW)])


def _entry(ids_flat, vals_pad, table, out,
           ids_v, vals_v, r0, r1, r2, r3, out_v, s0, s1, s2, s3):
    _body(ids_flat, vals_pad, table, out,
          ids_v, vals_v, (r0, r1, r2, r3), out_v, (s0, s1, s2, s3))


_sc_call = pl.kernel(
    _entry,
    out_type=jax.ShapeDtypeStruct((B, D), jnp.float32),
    mesh=plsc.VectorSubcoreMesh(core_axis_name="c", subcore_axis_name="s"),
    scratch_types=[
        pltpu.VMEM((BPW * HP,), jnp.int32),
        pltpu.VMEM((BPW, HP), jnp.float32),
        *[pltpu.VMEM((CIDX, D), jnp.float32) for _ in range(NBUF)],
        pltpu.VMEM((BPW, D), jnp.float32),
        *[pltpu.SemaphoreType.DMA for _ in range(NBUF)],
    ],
    compiler_params=pltpu.CompilerParams(use_tc_tiling_on_sc=False),
)


@jax.jit
def _run(ids, values, params):
    ids_p = jnp.zeros((B, HP), jnp.int32).at[:, :HIST].set(
        ids.astype(jnp.int32)).reshape(-1)
    vals_p = jnp.zeros((B, HP), jnp.float32).at[:, :HIST].set(values)
    return _sc_call(ids_p, vals_p, params)


def kernel(ids, values, params):
    return _run(ids, values, params)
